# baseline (speedup 1.0000x reference)
import functools

import jax
import jax.numpy as jnp
from jax import lax
from jax.experimental import pallas as pl
from jax.experimental.pallas import tpu as pltpu
from jax.experimental.pallas import tpu_sc as plsc

GAMMA = 24.0
EPSILON = 2.0
PI = 3.1415926235897933
CEN = 0.02

_SIN_C = (0.9999673025915434, -0.1666224038748874, 0.008316284383106321,
          -0.0001955960029411126, 2.527388131797967e-06,
          -1.588366235760752e-08)

CH = 64
RCH = 16
NLANE = 16


def _sin_poly(x):
    x2 = x * x
    p = jnp.float32(_SIN_C[5])
    for c in _SIN_C[4::-1]:
        p = p * x2 + jnp.float32(c)
    return x * p


def _sc_gather(nw, tpw, hpw, dim, ne, nr, ecomb, erad, rcomb,
               hidx, ridx, nidx):
    nneg_rows = nidx.shape[0]
    nb = hidx.shape[0]
    n_chunks = tpw // CH
    h_chunks = hpw // CH
    f32 = jnp.float32
    i32 = jnp.int32

    mesh = plsc.VectorSubcoreMesh(core_axis_name="c", subcore_axis_name="s")

    out_type = [
        jax.ShapeDtypeStruct((nneg_rows, dim), i32),
        jax.ShapeDtypeStruct((nneg_rows,), f32),
        jax.ShapeDtypeStruct((nb, dim), i32),
        jax.ShapeDtypeStruct((nb,), f32),
        jax.ShapeDtypeStruct((nb, 4 * dim), f32),
    ]

    @functools.partial(
        pl.kernel,
        out_type=out_type,
        mesh=mesh,
        compiler_params=pltpu.CompilerParams(needs_layout_passes=False),
        scratch_types=[
            pltpu.VMEM((ne,), f32),
            pltpu.VMEM((CH,), i32),
            pltpu.VMEM((CH,), i32),
            pltpu.VMEM((CH, dim), i32),
            pltpu.VMEM((CH, dim), i32),
            pltpu.VMEM((CH,), f32),
            pltpu.VMEM((CH,), f32),
            pltpu.VMEM((RCH, 4 * dim), f32),
            pltpu.SemaphoreType.DMA,
            pltpu.SemaphoreType.DMA,
            pltpu.SemaphoreType.DMA,
            pltpu.SemaphoreType.DMA,
        ],
    )
    def gather_kernel(ecomb_h, erad_h, rcomb_h, hidx_h, ridx_h, nidx_h,
                      tcomb_o, trad_o, hcomb_o, hrad_o, rcomb_o,
                      erad_v, idxa, idxb, rowa, rowb, rada, radb, relbuf,
                      gsa, gsb, wsa, wsb):
        wid = lax.axis_index("s") * 2 + lax.axis_index("c")
        pltpu.sync_copy(erad_h, erad_v)

        def rad_gather(idx_v, rad_v):
            for k in range(CH // NLANE):
                iv = idx_v[pl.ds(k * NLANE, NLANE)]
                rad_v[pl.ds(k * NLANE, NLANE)] = plsc.load_gather(erad_v,
                                                                  [iv])

        def chunked_rows(base, nck, idx_h, comb_o, rad_o):
            pltpu.sync_copy(idx_h.at[pl.ds(base, CH)], idxa)
            pltpu.async_copy(ecomb_h.at[idxa], rowa, gsa)
            pltpu.sync_copy(idx_h.at[pl.ds(base + CH, CH)], idxb)
            pltpu.async_copy(ecomb_h.at[idxb], rowb, gsb)

            def consume_refill(c, idx_v, row_v, rad_v, gs, ws):
                off = base + c * CH
                pltpu.make_async_copy(ecomb_h.at[idx_v], row_v, gs).wait()
                rad_gather(idx_v, rad_v)
                pltpu.async_copy(row_v, comb_o.at[pl.ds(off, CH)], ws)
                pltpu.async_copy(rad_v, rad_o.at[pl.ds(off, CH)], ws)

                @pl.when(c + 2 < nck)
                def _():
                    pltpu.make_async_copy(
                        row_v, comb_o.at[pl.ds(off, CH)], ws).wait()
                    pltpu.make_async_copy(
                        rad_v, rad_o.at[pl.ds(off, CH)], ws).wait()
                    off2 = base + (c + 2) * CH
                    pltpu.sync_copy(idx_h.at[pl.ds(off2, CH)], idx_v)
                    pltpu.async_copy(ecomb_h.at[idx_v], row_v, gs)

            def pair(p, carry):
                consume_refill(2 * p, idxa, rowa, rada, gsa, wsa)
                consume_refill(2 * p + 1, idxb, rowb, radb, gsb, wsb)
                return carry

            lax.fori_loop(0, nck // 2, pair, 0)
            pltpu.make_async_copy(rowa, comb_o.at[pl.ds(0, CH)], wsa).wait()
            pltpu.make_async_copy(rada, rad_o.at[pl.ds(0, CH)], wsa).wait()
            pltpu.make_async_copy(rowb, comb_o.at[pl.ds(0, CH)], wsb).wait()
            pltpu.make_async_copy(radb, rad_o.at[pl.ds(0, CH)], wsb).wait()

        chunked_rows(wid * tpw, n_chunks, nidx_h, tcomb_o, trad_o)
        chunked_rows(wid * hpw, h_chunks, hidx_h, hcomb_o, hrad_o)

        rbase = wid * hpw
        for c in range(hpw // CH):
            pltpu.sync_copy(ridx_h.at[pl.ds(rbase + c * CH, CH)], idxa)
            for s in range(CH // RCH):
                iv = idxa.at[pl.ds(s * RCH, RCH)]
                cg = pltpu.async_copy(rcomb_h.at[iv], relbuf, gsa)
                cg.wait()
                off = rbase + c * CH + s * RCH
                pltpu.sync_copy(relbuf, rcomb_o.at[pl.ds(off, RCH)])

    return gather_kernel(ecomb, erad, rcomb, hidx, ridx, nidx)


def _tc_score(erange, hcomb, hrad, rcombg, mod_weight, phase_weight,
              tcomb, trad):
    b, nneg, _ = tcomb.shape
    dim = hcomb.shape[1] // 2
    bq = 32
    inv_er = 1.0 / erange
    f32 = jnp.float32

    def body(mw_r, pw_r, hcomb_r, hrad_r, rcomb_r, tcomb_r, trad_r, out_r):
        mw = mw_r[0, 0]
        pw = pw_r[0, 0]
        hc = hcomb_r[...]
        hmod = hc[:, :dim].astype(f32)
        hph = hc[:, dim:].astype(f32)
        rc = rcomb_r[...]
        rm = jnp.abs(rc[:, :dim])
        rb = jnp.minimum(rc[:, dim:2 * dim], 1.0)
        rb = jnp.where(rb < -rm, -rm, rb)
        rph = rc[:, 2 * dim:3 * dim]
        rrad = rc[:, 3 * dim:3 * dim + 1]
        mod_e = hmod * (rm + rb)
        ph_half = hph + rph
        rad_e = jnp.abs(hrad_r[...] * inv_er) * jnp.abs(rrad)
        scale = 1.0 - rb

        tc = tcomb_r[...]
        tmod = tc[:, :, :dim].astype(f32)
        tph = tc[:, :, dim:].astype(f32)
        md = mod_e[:, None, :] - tmod * scale[:, None, :]
        mod_dist = jnp.sqrt(jnp.sum(md * md, axis=-1))
        pd = ph_half[:, None, :] - tph
        phase_dist = jnp.sum(jnp.abs(_sin_poly(pd)), axis=-1)
        rad_dist = jnp.abs(rad_e + jnp.abs(trad_r[...] * inv_er))
        out_r[...] = GAMMA - (mw * mod_dist + pw * phase_dist
                              - CEN * rad_dist)

    smem = pl.BlockSpec(memory_space=pltpu.SMEM)
    return pl.pallas_call(
        body,
        grid=(b // bq,),
        in_specs=[
            smem,
            smem,
            pl.BlockSpec((bq, 2 * dim), lambda i: (i, 0)),
            pl.BlockSpec((bq, 1), lambda i: (i, 0)),
            pl.BlockSpec((bq, 4 * dim), lambda i: (i, 0)),
            pl.BlockSpec((bq, nneg, 2 * dim), lambda i: (i, 0, 0)),
            pl.BlockSpec((bq, nneg), lambda i: (i, 0)),
        ],
        out_specs=pl.BlockSpec((bq, nneg), lambda i: (i, 0)),
        out_shape=jax.ShapeDtypeStruct((b, nneg), jnp.float32),
    )(mod_weight, phase_weight, hcomb, hrad, rcombg, tcomb, trad)


def kernel(entity_mod, entity_phase, entity_radius, relation_mod,
           relation_phase, relation_bias, relation_radius, mod_weight,
           phase_weight, head_idx, rel_idx, neg_idx):
    b, nneg = neg_idx.shape
    dim = entity_mod.shape[1]
    ne = entity_mod.shape[0]
    nr = relation_mod.shape[0]
    erange = (GAMMA + EPSILON) / dim
    half_inv = PI / erange * 0.5

    nw = 32
    tpw = (b * nneg) // nw
    hpw = b // nw
    assert tpw % (2 * CH) == 0 and hpw % (2 * CH) == 0

    bf16 = jnp.bfloat16
    ecomb16 = jnp.concatenate(
        [entity_mod.astype(bf16), (entity_phase * half_inv).astype(bf16)],
        axis=1)
    ecomb = lax.bitcast_convert_type(
        ecomb16.reshape(ne, dim, 2), jnp.int32)
    rcomb = jnp.concatenate(
        [relation_mod, relation_bias, relation_phase * half_inv,
         relation_radius,
         jnp.zeros((nr, dim - 1), jnp.float32)], axis=1)
    erad = entity_radius.reshape(-1)
    nidx = neg_idx.reshape(-1)

    tcomb, trad, hcomb, hrad, rcombg = _sc_gather(
        nw, tpw, hpw, dim, ne, nr, ecomb, erad, rcomb,
        head_idx, rel_idx, nidx)

    tcomb16 = lax.bitcast_convert_type(tcomb, bf16).reshape(
        b, nneg, 2 * dim)
    hcomb16 = lax.bitcast_convert_type(hcomb, bf16).reshape(b, 2 * dim)
    trad = trad.reshape(b, nneg)
    hrad = hrad[:, None]

    return _tc_score(erange, hcomb16, hrad, rcombg, mod_weight,
                     phase_weight, tcomb16, trad)


# speedup vs baseline: 5.0855x; 5.0855x over previous
import functools

import jax
import jax.numpy as jnp
from jax import lax
from jax.experimental import pallas as pl
from jax.experimental.pallas import tpu as pltpu
from jax.experimental.pallas import tpu_sc as plsc

GAMMA = 24.0
EPSILON = 2.0
PI = 3.1415926235897933
CEN = 0.02

_SIN_C = (0.9999673025915434, -0.1666224038748874, 0.008316284383106321,
          -0.0001955960029411126, 2.527388131797967e-06,
          -1.588366235760752e-08)

CH = 64
RCH = 16
NLANE = 16


def _sin_poly(x):
    x2 = x * x
    p = jnp.float32(_SIN_C[5])
    for c in _SIN_C[4::-1]:
        p = p * x2 + jnp.float32(c)
    return x * p


def _bf16_bits(x):
    return lax.bitcast_convert_type(
        x.astype(jnp.bfloat16).astype(jnp.float32), jnp.uint32)


def _tc_pack_entity(half_inv, emod, ephase):
    ne, dim = emod.shape
    br = 2000
    assert ne % br == 0

    def body(m_r, p_r, out_r):
        m = _bf16_bits(m_r[...])
        p = _bf16_bits(p_r[...] * half_inv)
        out_r[...] = lax.bitcast_convert_type(p | (m >> 16), jnp.int32)

    return pl.pallas_call(
        body,
        grid=(ne // br,),
        in_specs=[pl.BlockSpec((br, dim), lambda i: (i, 0)),
                  pl.BlockSpec((br, dim), lambda i: (i, 0))],
        out_specs=pl.BlockSpec((br, dim), lambda i: (i, 0)),
        out_shape=jax.ShapeDtypeStruct((ne, dim), jnp.int32),
    )(emod, ephase)


def _tc_pack_relation(half_inv, rmod, rbias, rphase, rrad):
    nr, dim = rmod.shape

    def body(m_r, b_r, p_r, r_r, out_r):
        out_r[:, :dim] = m_r[...]
        out_r[:, dim:2 * dim] = b_r[...]
        out_r[:, 2 * dim:3 * dim] = p_r[...] * half_inv
        out_r[:, 3 * dim:] = jnp.broadcast_to(r_r[...], (nr, dim))

    return pl.pallas_call(
        body,
        out_shape=jax.ShapeDtypeStruct((nr, 4 * dim), jnp.float32),
    )(rmod, rbias, rphase, rrad)


def _sc_gather(nw, tpw, hpw, dim, ne, nr, ecomb, erad, rcomb,
               hidx, ridx, nidx):
    nneg_rows = nidx.shape[0]
    nb = hidx.shape[0]
    n_chunks = tpw // CH
    h_chunks = hpw // CH
    f32 = jnp.float32
    i32 = jnp.int32

    mesh = plsc.VectorSubcoreMesh(core_axis_name="c", subcore_axis_name="s")

    out_type = [
        jax.ShapeDtypeStruct((nneg_rows, dim), i32),
        jax.ShapeDtypeStruct((nneg_rows,), f32),
        jax.ShapeDtypeStruct((nb, dim), i32),
        jax.ShapeDtypeStruct((nb,), f32),
        jax.ShapeDtypeStruct((nb, 4 * dim), f32),
    ]

    @functools.partial(
        pl.kernel,
        out_type=out_type,
        mesh=mesh,
        compiler_params=pltpu.CompilerParams(needs_layout_passes=False),
        scratch_types=[
            pltpu.VMEM((ne,), f32),
            pltpu.VMEM((CH,), i32),
            pltpu.VMEM((CH,), i32),
            pltpu.VMEM((CH, dim), i32),
            pltpu.VMEM((CH, dim), i32),
            pltpu.VMEM((CH,), f32),
            pltpu.VMEM((CH,), f32),
            pltpu.VMEM((RCH, 4 * dim), f32),
            pltpu.SemaphoreType.DMA,
            pltpu.SemaphoreType.DMA,
            pltpu.SemaphoreType.DMA,
            pltpu.SemaphoreType.DMA,
        ],
    )
    def gather_kernel(ecomb_h, erad_h, rcomb_h, hidx_h, ridx_h, nidx_h,
                      tcomb_o, trad_o, hcomb_o, hrad_o, rcomb_o,
                      erad_v, idxa, idxb, rowa, rowb, rada, radb, relbuf,
                      gsa, gsb, wsa, wsb):
        wid = lax.axis_index("s") * 2 + lax.axis_index("c")
        pltpu.sync_copy(erad_h, erad_v)

        def rad_gather(idx_v, rad_v):
            for k in range(CH // NLANE):
                iv = idx_v[pl.ds(k * NLANE, NLANE)]
                rad_v[pl.ds(k * NLANE, NLANE)] = plsc.load_gather(erad_v,
                                                                  [iv])

        def chunked_rows(base, nck, idx_h, comb_o, rad_o):
            pltpu.sync_copy(idx_h.at[pl.ds(base, CH)], idxa)
            pltpu.async_copy(ecomb_h.at[idxa], rowa, gsa)
            pltpu.sync_copy(idx_h.at[pl.ds(base + CH, CH)], idxb)
            pltpu.async_copy(ecomb_h.at[idxb], rowb, gsb)

            def consume_refill(c, idx_v, row_v, rad_v, gs, ws):
                off = base + c * CH
                pltpu.make_async_copy(ecomb_h.at[idx_v], row_v, gs).wait()
                rad_gather(idx_v, rad_v)
                pltpu.async_copy(row_v, comb_o.at[pl.ds(off, CH)], ws)
                pltpu.async_copy(rad_v, rad_o.at[pl.ds(off, CH)], ws)

                @pl.when(c + 2 < nck)
                def _():
                    pltpu.make_async_copy(
                        row_v, comb_o.at[pl.ds(off, CH)], ws).wait()
                    pltpu.make_async_copy(
                        rad_v, rad_o.at[pl.ds(off, CH)], ws).wait()
                    off2 = base + (c + 2) * CH
                    pltpu.sync_copy(idx_h.at[pl.ds(off2, CH)], idx_v)
                    pltpu.async_copy(ecomb_h.at[idx_v], row_v, gs)

            def pair(p, carry):
                consume_refill(2 * p, idxa, rowa, rada, gsa, wsa)
                consume_refill(2 * p + 1, idxb, rowb, radb, gsb, wsb)
                return carry

            lax.fori_loop(0, nck // 2, pair, 0)
            pltpu.make_async_copy(rowa, comb_o.at[pl.ds(0, CH)], wsa).wait()
            pltpu.make_async_copy(rada, rad_o.at[pl.ds(0, CH)], wsa).wait()
            pltpu.make_async_copy(rowb, comb_o.at[pl.ds(0, CH)], wsb).wait()
            pltpu.make_async_copy(radb, rad_o.at[pl.ds(0, CH)], wsb).wait()

        chunked_rows(wid * tpw, n_chunks, nidx_h, tcomb_o, trad_o)
        chunked_rows(wid * hpw, h_chunks, hidx_h, hcomb_o, hrad_o)

        rbase = wid * hpw
        for c in range(hpw // CH):
            pltpu.sync_copy(ridx_h.at[pl.ds(rbase + c * CH, CH)], idxa)
            for s in range(CH // RCH):
                iv = idxa.at[pl.ds(s * RCH, RCH)]
                cg = pltpu.async_copy(rcomb_h.at[iv], relbuf, gsa)
                cg.wait()
                off = rbase + c * CH + s * RCH
                pltpu.sync_copy(relbuf, rcomb_o.at[pl.ds(off, RCH)])

    return gather_kernel(ecomb, erad, rcomb, hidx, ridx, nidx)


def _tc_score(erange, hcomb, hrad, rcombg, mod_weight, phase_weight,
              tcomb, trad):
    b, nneg, dim = tcomb.shape
    bq = 32
    inv_er = 1.0 / erange
    f32 = jnp.float32
    u32 = jnp.uint32

    def unpack(x_i32):
        u = lax.bitcast_convert_type(x_i32, u32)
        lo = lax.bitcast_convert_type(u << 16, f32)
        hi = lax.bitcast_convert_type(u & jnp.uint32(0xFFFF0000),
                                      f32)
        return lo, hi

    def body(mw_r, pw_r, hcomb_r, hrad_r, rcomb_r, tcomb_r, trad_r, out_r):
        mw = mw_r[0, 0]
        pw = pw_r[0, 0]
        hmod, hph = unpack(hcomb_r[...])
        rc = rcomb_r[...]
        rm = jnp.abs(rc[:, :dim])
        rb = jnp.minimum(rc[:, dim:2 * dim], 1.0)
        rb = jnp.where(rb < -rm, -rm, rb)
        rph = rc[:, 2 * dim:3 * dim]
        rrad = rc[:, 3 * dim:3 * dim + 1]
        mod_e = hmod * (rm + rb)
        ph_half = hph + rph
        rad_e = jnp.abs(hrad_r[...] * inv_er) * jnp.abs(rrad)
        scale = 1.0 - rb

        tmod, tph = unpack(tcomb_r[...])
        md = mod_e[:, None, :] - tmod * scale[:, None, :]
        mod_dist = jnp.sqrt(jnp.sum(md * md, axis=-1))
        pd = ph_half[:, None, :] - tph
        phase_dist = jnp.sum(jnp.abs(_sin_poly(pd)), axis=-1)
        rad_dist = jnp.abs(rad_e + jnp.abs(trad_r[...] * inv_er))
        out_r[...] = GAMMA - (mw * mod_dist + pw * phase_dist
                              - CEN * rad_dist)

    smem = pl.BlockSpec(memory_space=pltpu.SMEM)
    return pl.pallas_call(
        body,
        grid=(b // bq,),
        in_specs=[
            smem,
            smem,
            pl.BlockSpec((bq, dim), lambda i: (i, 0)),
            pl.BlockSpec((bq, 1), lambda i: (i, 0)),
            pl.BlockSpec((bq, 4 * dim), lambda i: (i, 0)),
            pl.BlockSpec((bq, nneg, dim), lambda i: (i, 0, 0)),
            pl.BlockSpec((bq, nneg), lambda i: (i, 0)),
        ],
        out_specs=pl.BlockSpec((bq, nneg), lambda i: (i, 0)),
        out_shape=jax.ShapeDtypeStruct((b, nneg), jnp.float32),
    )(mod_weight, phase_weight, hcomb, hrad, rcombg, tcomb, trad)


def kernel(entity_mod, entity_phase, entity_radius, relation_mod,
           relation_phase, relation_bias, relation_radius, mod_weight,
           phase_weight, head_idx, rel_idx, neg_idx):
    b, nneg = neg_idx.shape
    dim = entity_mod.shape[1]
    ne = entity_mod.shape[0]
    nr = relation_mod.shape[0]
    erange = (GAMMA + EPSILON) / dim
    half_inv = PI / erange * 0.5

    nw = 32
    tpw = (b * nneg) // nw
    hpw = b // nw
    assert tpw % (2 * CH) == 0 and hpw % (2 * CH) == 0

    ecomb = _tc_pack_entity(half_inv, entity_mod, entity_phase)
    rcomb = _tc_pack_relation(half_inv, relation_mod, relation_bias,
                              relation_phase, relation_radius)
    erad = entity_radius.reshape(-1)
    nidx = neg_idx.reshape(-1)

    tcomb, trad, hcomb, hrad, rcombg = _sc_gather(
        nw, tpw, hpw, dim, ne, nr, ecomb, erad, rcomb,
        head_idx, rel_idx, nidx)

    tcomb = tcomb.reshape(b, nneg, dim)
    trad = trad.reshape(b, nneg)
    hrad = hrad[:, None]

    return _tc_score(erange, hcomb, hrad, rcombg, mod_weight,
                     phase_weight, tcomb, trad)
